# Initial kernel scaffold; baseline (speedup 1.0000x reference)
import jax
import jax.numpy as jnp
from jax.experimental import pallas as pl


def kernel(x, emb_table, fc_w, fc_b):
    raise NotImplementedError("write your pallas kernel here")


# speedup vs baseline: 1.7409x; 1.7409x over previous
import functools

import jax
import jax.numpy as jnp
from jax import lax
from jax.experimental import pallas as pl
from jax.experimental.pallas import tpu as pltpu
from jax.experimental.pallas import tpu_sc as plsc

EMBED_DIM = 64
OUT_DIM = 1280

NUM_CORES = 2
NUM_SUBCORES = 16
NW = NUM_CORES * NUM_SUBCORES
CHUNK = 128


def _sc_gather(table, idx3):
  n_chunks = idx3.shape[1]
  b_per_w = n_chunks * CHUNK
  total = NW * b_per_w
  mesh = plsc.VectorSubcoreMesh(core_axis_name="c", subcore_axis_name="s")

  @functools.partial(
      pl.kernel,
      out_type=jax.ShapeDtypeStruct((total, EMBED_DIM), jnp.float32),
      mesh=mesh,
      scratch_types=[
          pltpu.VMEM((n_chunks, CHUNK), jnp.int32),
          pltpu.VMEM((CHUNK, EMBED_DIM), jnp.float32),
          pltpu.SemaphoreType.DMA,
      ],
      compiler_params=pltpu.CompilerParams(use_tc_tiling_on_sc=False),
  )
  def gather_kernel(table_hbm, idx_hbm, out_hbm, idx_v, rows_v, gsem):
    wid = lax.axis_index("s") * NUM_CORES + lax.axis_index("c")
    pltpu.sync_copy(idx_hbm.at[wid], idx_v)

    def body(j, carry):
      base = wid * b_per_w + j * CHUNK
      pltpu.async_copy(table_hbm.at[idx_v.at[j]], rows_v, gsem).wait()
      pltpu.sync_copy(rows_v, out_hbm.at[pl.ds(base, CHUNK)])
      return carry

    lax.fori_loop(0, n_chunks, body, 0)

  return gather_kernel(table, idx3)


def _mm_body(a_ref, w_ref, b_ref, o_ref):
  o_ref[...] = (
      jnp.dot(a_ref[...], w_ref[...], preferred_element_type=jnp.float32)
      + b_ref[...]
  )


def _tc_matmul(a, w, b, block_m):
  m = a.shape[0]
  return pl.pallas_call(
      _mm_body,
      grid=(m // block_m,),
      in_specs=[
          pl.BlockSpec((block_m, EMBED_DIM), lambda i: (i, 0)),
          pl.BlockSpec((EMBED_DIM, OUT_DIM), lambda i: (0, 0)),
          pl.BlockSpec((1, OUT_DIM), lambda i: (0, 0)),
      ],
      out_specs=pl.BlockSpec((block_m, OUT_DIM), lambda i: (i, 0)),
      out_shape=jax.ShapeDtypeStruct((m, OUT_DIM), jnp.float32),
      compiler_params=pltpu.CompilerParams(
          dimension_semantics=("arbitrary",),
      ),
  )(a, w, b.reshape(1, OUT_DIM))


def kernel(x, emb_table, fc_w, fc_b):
  batch, hist = x.shape
  flat = x.reshape(-1).astype(jnp.int32)
  idx3 = flat.reshape(NW, -1, CHUNK)
  gathered = _sc_gather(emb_table, idx3)
  out = _tc_matmul(gathered, fc_w, fc_b, block_m=1024)
  return out.reshape(batch, hist, OUT_DIM)


# speedup vs baseline: 1.8316x; 1.0521x over previous
import functools

import jax
import jax.numpy as jnp
from jax import lax
from jax.experimental import pallas as pl
from jax.experimental.pallas import tpu as pltpu
from jax.experimental.pallas import tpu_sc as plsc

EMBED_DIM = 64
OUT_DIM = 1280

NUM_CORES = 2
NUM_SUBCORES = 16
NW = NUM_CORES * NUM_SUBCORES
CHUNK = 128
GPB = 4
MACRO = CHUNK * GPB


def _sc_gather(table, idx3):
  n_chunks = idx3.shape[1]
  b_per_w = n_chunks * CHUNK
  total = NW * b_per_w
  n_macro = b_per_w // MACRO
  assert n_macro % 2 == 0 and n_chunks % GPB == 0
  n2 = n_macro // 2
  mesh = plsc.VectorSubcoreMesh(core_axis_name="c", subcore_axis_name="s")

  @functools.partial(
      pl.kernel,
      out_type=jax.ShapeDtypeStruct((total, EMBED_DIM), jnp.float32),
      mesh=mesh,
      scratch_types=[
          pltpu.VMEM((n_chunks, CHUNK), jnp.int32),
          pltpu.VMEM((MACRO, EMBED_DIM), jnp.float32),
          pltpu.VMEM((MACRO, EMBED_DIM), jnp.float32),
          pltpu.SemaphoreType.DMA,
          pltpu.SemaphoreType.DMA,
          pltpu.SemaphoreType.DMA,
          pltpu.SemaphoreType.DMA,
      ],
      compiler_params=pltpu.CompilerParams(use_tc_tiling_on_sc=False),
  )
  def gather_kernel(table_hbm, idx_hbm, out_hbm, idx_v, s0, s1, g0, g1, w0, w1):
    wid = lax.axis_index("s") * NUM_CORES + lax.axis_index("c")
    base_w = wid * b_per_w
    dummy = out_hbm.at[pl.ds(0, MACRO)]
    pltpu.sync_copy(idx_hbm.at[wid], idx_v)

    for k in range(GPB):
      pltpu.async_copy(table_hbm.at[idx_v.at[k]], s0.at[pl.ds(k * CHUNK, CHUNK)], g0)

    def body(i, carry):
      m0 = 2 * i
      m1 = m0 + 1

      @pl.when(i > 0)
      def _():
        pltpu.make_async_copy(s1, dummy, w1).wait()

      for k in range(GPB):
        pltpu.async_copy(
            table_hbm.at[idx_v.at[m1 * GPB + k]],
            s1.at[pl.ds(k * CHUNK, CHUNK)], g1)

      pltpu.make_async_copy(dummy, s0, g0).wait()
      pltpu.async_copy(s0, out_hbm.at[pl.ds(base_w + m0 * MACRO, MACRO)], w0)

      @pl.when(i < n2 - 1)
      def _():
        pltpu.make_async_copy(s0, dummy, w0).wait()
        for k in range(GPB):
          pltpu.async_copy(
              table_hbm.at[idx_v.at[(m0 + 2) * GPB + k]],
              s0.at[pl.ds(k * CHUNK, CHUNK)], g0)

      pltpu.make_async_copy(dummy, s1, g1).wait()
      pltpu.async_copy(s1, out_hbm.at[pl.ds(base_w + m1 * MACRO, MACRO)], w1)
      return carry

    lax.fori_loop(0, n2, body, 0)
    pltpu.make_async_copy(s0, dummy, w0).wait()
    pltpu.make_async_copy(s1, dummy, w1).wait()

  return gather_kernel(table, idx3)


def _mm_body(a_ref, w_ref, b_ref, o_ref):
  o_ref[...] = (
      jnp.dot(a_ref[...], w_ref[...], preferred_element_type=jnp.float32)
      + b_ref[...]
  )


def _tc_matmul(a, w, b, block_m):
  m = a.shape[0]
  return pl.pallas_call(
      _mm_body,
      grid=(m // block_m,),
      in_specs=[
          pl.BlockSpec((block_m, EMBED_DIM), lambda i: (i, 0)),
          pl.BlockSpec((EMBED_DIM, OUT_DIM), lambda i: (0, 0)),
          pl.BlockSpec((1, OUT_DIM), lambda i: (0, 0)),
      ],
      out_specs=pl.BlockSpec((block_m, OUT_DIM), lambda i: (i, 0)),
      out_shape=jax.ShapeDtypeStruct((m, OUT_DIM), jnp.float32),
      compiler_params=pltpu.CompilerParams(
          dimension_semantics=("arbitrary",),
      ),
  )(a, w, b.reshape(1, OUT_DIM))


def kernel(x, emb_table, fc_w, fc_b):
  batch, hist = x.shape
  flat = x.reshape(-1).astype(jnp.int32)
  idx3 = flat.reshape(NW, -1, CHUNK)
  gathered = _sc_gather(emb_table, idx3)
  out = _tc_matmul(gathered, fc_w, fc_b, block_m=1024)
  return out.reshape(batch, hist, OUT_DIM)
